# Initial kernel scaffold; baseline (speedup 1.0000x reference)
import functools

import jax
import jax.numpy as jnp
from jax import lax
from jax.experimental import pallas as pl
from jax.experimental.pallas import tpu as pltpu
from jax.experimental.pallas import tpu_sc as plsc

N = 10000
E = 320000
H = 128
G = 64

NC = 2
NS = 16
NW = NC * NS
EDGES_PER_TILE = E // NW
CHUNK = 80
NCHUNKS = EDGES_PER_TILE // CHUNK
ROWS_PER_SUB = N // NS

BLK = 1000
NBLK = N // BLK


def _sc_agg(h, src, dst, zeros):
    mesh = plsc.VectorSubcoreMesh(core_axis_name="c", subcore_axis_name="s")

    @functools.partial(
        pl.kernel,
        mesh=mesh,
        out_type=jax.ShapeDtypeStruct((NC, N, H), jnp.float32),
        scratch_types=[
            pltpu.VMEM((CHUNK,), jnp.int32),
            pltpu.VMEM((CHUNK,), jnp.int32),
            pltpu.VMEM((CHUNK, H), jnp.float32),
            pltpu.VMEM_SHARED((N, H), jnp.float32),
            pltpu.SemaphoreType.DMA,
        ],
    )
    def k(h_hbm, src_hbm, dst_hbm, zeros_hbm, out_hbm, src_v, dst_v, rows_v,
          acc, sem):
        cid = lax.axis_index("c")
        sid = lax.axis_index("s")
        wid = sid * NC + cid
        row0 = sid * ROWS_PER_SUB
        pltpu.sync_copy(zeros_hbm.at[pl.ds(row0, ROWS_PER_SUB)],
                        acc.at[pl.ds(row0, ROWS_PER_SUB)])
        plsc.subcore_barrier()

        base = wid * EDGES_PER_TILE

        @pl.loop(0, NCHUNKS)
        def _(c):
            off = base + c * CHUNK
            pltpu.sync_copy(src_hbm.at[pl.ds(off, CHUNK)], src_v)
            pltpu.sync_copy(dst_hbm.at[pl.ds(off, CHUNK)], dst_v)
            pltpu.async_copy(h_hbm.at[src_v], rows_v, sem).wait()
            pltpu.sync_copy(rows_v, acc.at[dst_v], add=True)

        plsc.subcore_barrier()
        pltpu.sync_copy(acc.at[pl.ds(row0, ROWS_PER_SUB)],
                        out_hbm.at[cid, pl.ds(row0, ROWS_PER_SUB)])

    return k(h, src, dst, zeros)


def _tc_mlp(h, a0, a1, W1, b1, W2, b2):
    def body(h_ref, a0_ref, a1_ref, w1_ref, b1_ref, w2_ref, b2_ref,
             m_ref, s_ref):
        i = pl.program_id(0)
        x = h_ref[...] + a0_ref[...] + a1_ref[...]
        t = jnp.dot(x, w1_ref[...], preferred_element_type=jnp.float32)
        t = jnp.maximum(t + b1_ref[...], 0.0)
        t = jnp.dot(t, w2_ref[...], preferred_element_type=jnp.float32)
        t = t + b2_ref[...]
        m = jnp.where(t > 0.0, t, jnp.expm1(t))
        m_ref[...] = m
        srow = jnp.sum(m, axis=0, keepdims=True)
        sqrow = jnp.sum(m * m, axis=0, keepdims=True)
        stats = jnp.concatenate(
            [srow, sqrow, jnp.zeros((6, H), jnp.float32)], axis=0)

        @pl.when(i == 0)
        def _():
            s_ref[...] = stats

        @pl.when(i != 0)
        def _():
            s_ref[...] += stats

    return pl.pallas_call(
        body,
        grid=(NBLK,),
        in_specs=[
            pl.BlockSpec((BLK, H), lambda i: (i, 0)),
            pl.BlockSpec((BLK, H), lambda i: (i, 0)),
            pl.BlockSpec((BLK, H), lambda i: (i, 0)),
            pl.BlockSpec((H, H), lambda i: (0, 0)),
            pl.BlockSpec((1, H), lambda i: (0, 0)),
            pl.BlockSpec((H, H), lambda i: (0, 0)),
            pl.BlockSpec((1, H), lambda i: (0, 0)),
        ],
        out_specs=[
            pl.BlockSpec((BLK, H), lambda i: (i, 0)),
            pl.BlockSpec((8, H), lambda i: (0, 0)),
        ],
        out_shape=[
            jax.ShapeDtypeStruct((N, H), jnp.float32),
            jax.ShapeDtypeStruct((8, H), jnp.float32),
        ],
    )(h, a0, a1, W1, b1.reshape(1, H), W2, b2.reshape(1, H))


def _tc_bn_pool(m, stats, gamma, beta, batch3):
    def body(m_ref, s_ref, g_ref, b_ref, bt_ref, h_ref, p_ref):
        i = pl.program_id(0)
        mu = s_ref[0, :] * (1.0 / N)
        var = s_ref[1, :] * (1.0 / N) - mu * mu
        inv = lax.rsqrt(var + 1e-5)
        scale = inv * g_ref[0, :]
        shift = b_ref[0, :] - mu * scale
        hh = m_ref[...] * scale[None, :] + shift[None, :]
        h_ref[...] = hh
        bt = bt_ref[0, 0, :]
        onehot = (bt[:, None] == lax.broadcasted_iota(jnp.int32, (BLK, G), 1)
                  ).astype(jnp.float32)
        pool = lax.dot_general(onehot, hh, (((0,), (0,)), ((), ())),
                               preferred_element_type=jnp.float32)

        @pl.when(i == 0)
        def _():
            p_ref[...] = pool

        @pl.when(i != 0)
        def _():
            p_ref[...] += pool

    return pl.pallas_call(
        body,
        grid=(NBLK,),
        in_specs=[
            pl.BlockSpec((BLK, H), lambda i: (i, 0)),
            pl.BlockSpec((8, H), lambda i: (0, 0)),
            pl.BlockSpec((1, H), lambda i: (0, 0)),
            pl.BlockSpec((1, H), lambda i: (0, 0)),
            pl.BlockSpec((1, 1, BLK), lambda i: (i, 0, 0)),
        ],
        out_specs=[
            pl.BlockSpec((BLK, H), lambda i: (i, 0)),
            pl.BlockSpec((G, H), lambda i: (0, 0)),
        ],
        out_shape=[
            jax.ShapeDtypeStruct((N, H), jnp.float32),
            jax.ShapeDtypeStruct((G, H), jnp.float32),
        ],
    )(m, stats, gamma.reshape(1, H), beta.reshape(1, H), batch3)


def kernel(x, edge_index, batch,
           W1_0, b1_0, W2_0, b2_0, gamma_0, beta_0,
           W1_1, b1_1, W2_1, b2_1, gamma_1, beta_1,
           W1_2, b1_2, W2_2, b2_2, gamma_2, beta_2):
    src = edge_index[0]
    dst = edge_index[1]
    zeros = jnp.zeros((N, H), jnp.float32)
    batch3 = batch.reshape(NBLK, 1, BLK)
    layers = [(W1_0, b1_0, W2_0, b2_0, gamma_0, beta_0),
              (W1_1, b1_1, W2_1, b2_1, gamma_1, beta_1),
              (W1_2, b1_2, W2_2, b2_2, gamma_2, beta_2)]
    h = x
    xs = []
    pools = []
    for (W1, b1, W2, b2, g, b) in layers:
        agg = _sc_agg(h, src, dst, zeros)
        m, stats = _tc_mlp(h, agg[0], agg[1], W1, b1, W2, b2)
        h, pool = _tc_bn_pool(m, stats, g, b, batch3)
        xs.append(h)
        pools.append(pool)
    return (jnp.concatenate(pools, axis=1), jnp.concatenate(xs, axis=1))


# speedup vs baseline: 4.4209x; 4.4209x over previous
import functools

import jax
import jax.numpy as jnp
from jax import lax
from jax.experimental import pallas as pl
from jax.experimental.pallas import tpu as pltpu
from jax.experimental.pallas import tpu_sc as plsc

N = 10000
E = 320000
H = 128
G = 64

NC = 2
NS = 16
NW = NC * NS
EDGES_PER_TILE = E // NW
CHUNK = 80
NCHUNKS = EDGES_PER_TILE // CHUNK
ROWS_PER_SUB = 632
N_PAD = NS * ROWS_PER_SUB

BLK = 1000
NBLK = N // BLK


def _sc_agg(h, src, dst, zeros):
    mesh = plsc.VectorSubcoreMesh(core_axis_name="c", subcore_axis_name="s")

    @functools.partial(
        pl.kernel,
        mesh=mesh,
        out_type=jax.ShapeDtypeStruct((NC, N_PAD, H), jnp.float32),
        scratch_types=[
            pltpu.VMEM((CHUNK,), jnp.int32),
            pltpu.VMEM((CHUNK,), jnp.int32),
            pltpu.VMEM((CHUNK, H), jnp.float32),
            pltpu.VMEM_SHARED((N_PAD, H), jnp.float32),
            pltpu.SemaphoreType.DMA,
        ],
    )
    def k(h_hbm, src_hbm, dst_hbm, zeros_hbm, out_hbm, src_v, dst_v, rows_v,
          acc, sem):
        cid = lax.axis_index("c")
        sid = lax.axis_index("s")
        wid = sid * NC + cid
        row0 = sid * ROWS_PER_SUB
        pltpu.sync_copy(zeros_hbm.at[pl.ds(row0, ROWS_PER_SUB)],
                        acc.at[pl.ds(row0, ROWS_PER_SUB)])
        plsc.subcore_barrier()

        base = wid * EDGES_PER_TILE

        @pl.loop(0, NCHUNKS)
        def _(c):
            off = base + c * CHUNK
            pltpu.sync_copy(src_hbm.at[pl.ds(off, CHUNK)], src_v)
            pltpu.sync_copy(dst_hbm.at[pl.ds(off, CHUNK)], dst_v)
            pltpu.async_copy(h_hbm.at[src_v], rows_v, sem).wait()
            pltpu.sync_copy(rows_v, acc.at[dst_v], add=True)

        plsc.subcore_barrier()
        pltpu.sync_copy(acc.at[pl.ds(row0, ROWS_PER_SUB)],
                        out_hbm.at[cid, pl.ds(row0, ROWS_PER_SUB)])

    return k(h, src, dst, zeros)


def _tc_mlp(h, a0, a1, W1, b1, W2, b2):
    def body(h_ref, a0_ref, a1_ref, w1_ref, b1_ref, w2_ref, b2_ref,
             m_ref, s_ref):
        i = pl.program_id(0)
        x = h_ref[...] + a0_ref[...] + a1_ref[...]
        t = jnp.dot(x, w1_ref[...], preferred_element_type=jnp.float32)
        t = jnp.maximum(t + b1_ref[...], 0.0)
        t = jnp.dot(t, w2_ref[...], preferred_element_type=jnp.float32)
        t = t + b2_ref[...]
        m = jnp.where(t > 0.0, t, jnp.exp(jnp.minimum(t, 0.0)) - 1.0)
        m_ref[...] = m
        srow = jnp.sum(m, axis=0, keepdims=True)
        sqrow = jnp.sum(m * m, axis=0, keepdims=True)
        stats = jnp.concatenate(
            [srow, sqrow, jnp.zeros((6, H), jnp.float32)], axis=0)

        @pl.when(i == 0)
        def _():
            s_ref[...] = stats

        @pl.when(i != 0)
        def _():
            s_ref[...] += stats

    return pl.pallas_call(
        body,
        grid=(NBLK,),
        in_specs=[
            pl.BlockSpec((BLK, H), lambda i: (i, 0)),
            pl.BlockSpec((BLK, H), lambda i: (i, 0)),
            pl.BlockSpec((BLK, H), lambda i: (i, 0)),
            pl.BlockSpec((H, H), lambda i: (0, 0)),
            pl.BlockSpec((1, H), lambda i: (0, 0)),
            pl.BlockSpec((H, H), lambda i: (0, 0)),
            pl.BlockSpec((1, H), lambda i: (0, 0)),
        ],
        out_specs=[
            pl.BlockSpec((BLK, H), lambda i: (i, 0)),
            pl.BlockSpec((8, H), lambda i: (0, 0)),
        ],
        out_shape=[
            jax.ShapeDtypeStruct((N, H), jnp.float32),
            jax.ShapeDtypeStruct((8, H), jnp.float32),
        ],
    )(h, a0, a1, W1, b1.reshape(1, H), W2, b2.reshape(1, H))


def _tc_bn_pool(m, stats, gamma, beta, batch3):
    def body(m_ref, s_ref, g_ref, b_ref, bt_ref, h_ref, p_ref):
        i = pl.program_id(0)
        mu = s_ref[0, :] * (1.0 / N)
        var = s_ref[1, :] * (1.0 / N) - mu * mu
        inv = lax.rsqrt(var + 1e-5)
        scale = inv * g_ref[0, :]
        shift = b_ref[0, :] - mu * scale
        hh = m_ref[...] * scale[None, :] + shift[None, :]
        h_ref[...] = hh
        bt = bt_ref[0, 0, :]
        onehot = (bt[:, None] == lax.broadcasted_iota(jnp.int32, (BLK, G), 1)
                  ).astype(jnp.float32)
        pool = lax.dot_general(onehot, hh, (((0,), (0,)), ((), ())),
                               preferred_element_type=jnp.float32)

        @pl.when(i == 0)
        def _():
            p_ref[...] = pool

        @pl.when(i != 0)
        def _():
            p_ref[...] += pool

    return pl.pallas_call(
        body,
        grid=(NBLK,),
        in_specs=[
            pl.BlockSpec((BLK, H), lambda i: (i, 0)),
            pl.BlockSpec((8, H), lambda i: (0, 0)),
            pl.BlockSpec((1, H), lambda i: (0, 0)),
            pl.BlockSpec((1, H), lambda i: (0, 0)),
            pl.BlockSpec((1, 1, BLK), lambda i: (i, 0, 0)),
        ],
        out_specs=[
            pl.BlockSpec((BLK, H), lambda i: (i, 0)),
            pl.BlockSpec((G, H), lambda i: (0, 0)),
        ],
        out_shape=[
            jax.ShapeDtypeStruct((N, H), jnp.float32),
            jax.ShapeDtypeStruct((G, H), jnp.float32),
        ],
    )(m, stats, gamma.reshape(1, H), beta.reshape(1, H), batch3)


def kernel(x, edge_index, batch,
           W1_0, b1_0, W2_0, b2_0, gamma_0, beta_0,
           W1_1, b1_1, W2_1, b2_1, gamma_1, beta_1,
           W1_2, b1_2, W2_2, b2_2, gamma_2, beta_2):
    src = edge_index[0]
    dst = edge_index[1]
    zeros = jnp.zeros((N_PAD, H), jnp.float32)
    batch3 = batch.reshape(NBLK, 1, BLK)
    layers = [(W1_0, b1_0, W2_0, b2_0, gamma_0, beta_0),
              (W1_1, b1_1, W2_1, b2_1, gamma_1, beta_1),
              (W1_2, b1_2, W2_2, b2_2, gamma_2, beta_2)]
    h = x
    xs = []
    pools = []
    for (W1, b1, W2, b2, g, b) in layers:
        agg = _sc_agg(h, src, dst, zeros)
        m, stats = _tc_mlp(h, agg[0], agg[1], W1, b1, W2, b2)
        h, pool = _tc_bn_pool(m, stats, g, b, batch3)
        xs.append(h)
        pools.append(pool)
    return (jnp.concatenate(pools, axis=1), jnp.concatenate(xs, axis=1))


# speedup vs baseline: 9.7001x; 2.1941x over previous
import functools

import jax
import jax.numpy as jnp
from jax import lax
from jax.experimental import pallas as pl
from jax.experimental.pallas import tpu as pltpu
from jax.experimental.pallas import tpu_sc as plsc

N = 10000
E = 320000
H = 128
G = 64

NC = 2
NS = 16
NW = NC * NS
EDGES_PER_TILE = E // NW
CHUNK = 80
NCHUNKS = EDGES_PER_TILE // CHUNK
ROWS_PER_SUB = 632
N_PAD = NS * ROWS_PER_SUB

BLK = 1000
NBLK = N // BLK


def _sc_agg(h, src, dst, zeros):
    mesh = plsc.VectorSubcoreMesh(core_axis_name="c", subcore_axis_name="s")

    NPAIRS = (NCHUNKS - 1) // 2

    @functools.partial(
        pl.kernel,
        mesh=mesh,
        out_type=jax.ShapeDtypeStruct((NC, N_PAD, H), jnp.float32),
        scratch_types=[
            pltpu.VMEM((EDGES_PER_TILE,), jnp.int32),
            pltpu.VMEM((NCHUNKS, CHUNK), jnp.int32),
            pltpu.VMEM((CHUNK, H), jnp.float32),
            pltpu.VMEM((CHUNK, H), jnp.float32),
            pltpu.VMEM_SHARED((N_PAD, H), jnp.float32),
            pltpu.SemaphoreType.DMA,
            pltpu.SemaphoreType.DMA,
            pltpu.SemaphoreType.DMA,
        ],
    )
    def k(h_hbm, src_hbm, dst_hbm, zeros_hbm, out_hbm, src_all, dst_all,
          rows0, rows1, acc, sem0, sem1, sem_idx):
        cid = lax.axis_index("c")
        sid = lax.axis_index("s")
        wid = sid * NC + cid
        row0 = sid * ROWS_PER_SUB
        pltpu.async_copy(src_hbm.at[wid], src_all, sem_idx)
        pltpu.async_copy(dst_hbm.at[wid], dst_all, sem_idx)
        pltpu.sync_copy(zeros_hbm.at[pl.ds(row0, ROWS_PER_SUB)],
                        acc.at[pl.ds(row0, ROWS_PER_SUB)])
        pltpu.make_async_copy(src_hbm.at[wid], src_all, sem_idx).wait()
        pltpu.make_async_copy(dst_hbm.at[wid], dst_all, sem_idx).wait()

        def gather_start(c, rows, sem):
            pltpu.async_copy(h_hbm.at[src_all.at[pl.ds(c * CHUNK, CHUNK)]],
                             rows, sem)

        def gather_wait(c, rows, sem):
            pltpu.make_async_copy(
                h_hbm.at[src_all.at[pl.ds(c * CHUNK, CHUNK)]], rows, sem
            ).wait()

        gather_start(0, rows0, sem0)
        plsc.subcore_barrier()

        @pl.loop(0, NPAIRS)
        def _(i):
            c = 2 * i
            gather_start(c + 1, rows1, sem1)
            gather_wait(c, rows0, sem0)
            pltpu.sync_copy(rows0, acc.at[dst_all.at[c]], add=True)
            gather_start(c + 2, rows0, sem0)
            gather_wait(c + 1, rows1, sem1)
            pltpu.sync_copy(rows1, acc.at[dst_all.at[c + 1]], add=True)

        gather_wait(NCHUNKS - 1, rows0, sem0)
        pltpu.sync_copy(rows0, acc.at[dst_all.at[NCHUNKS - 1]], add=True)

        plsc.subcore_barrier()
        pltpu.sync_copy(acc.at[pl.ds(row0, ROWS_PER_SUB)],
                        out_hbm.at[cid, pl.ds(row0, ROWS_PER_SUB)])

    return k(h, src, dst, zeros)


def _tc_mlp(h, a0, a1, W1, b1, W2, b2):
    def body(h_ref, a0_ref, a1_ref, w1_ref, b1_ref, w2_ref, b2_ref,
             m_ref, s_ref):
        i = pl.program_id(0)
        x = h_ref[...] + a0_ref[...] + a1_ref[...]
        t = jnp.dot(x, w1_ref[...], preferred_element_type=jnp.float32)
        t = jnp.maximum(t + b1_ref[...], 0.0)
        t = jnp.dot(t, w2_ref[...], preferred_element_type=jnp.float32)
        t = t + b2_ref[...]
        m = jnp.where(t > 0.0, t, jnp.exp(jnp.minimum(t, 0.0)) - 1.0)
        m_ref[...] = m
        srow = jnp.sum(m, axis=0, keepdims=True)
        sqrow = jnp.sum(m * m, axis=0, keepdims=True)
        stats = jnp.concatenate(
            [srow, sqrow, jnp.zeros((6, H), jnp.float32)], axis=0)

        @pl.when(i == 0)
        def _():
            s_ref[...] = stats

        @pl.when(i != 0)
        def _():
            s_ref[...] += stats

    return pl.pallas_call(
        body,
        grid=(NBLK,),
        in_specs=[
            pl.BlockSpec((BLK, H), lambda i: (i, 0)),
            pl.BlockSpec((BLK, H), lambda i: (i, 0)),
            pl.BlockSpec((BLK, H), lambda i: (i, 0)),
            pl.BlockSpec((H, H), lambda i: (0, 0)),
            pl.BlockSpec((1, H), lambda i: (0, 0)),
            pl.BlockSpec((H, H), lambda i: (0, 0)),
            pl.BlockSpec((1, H), lambda i: (0, 0)),
        ],
        out_specs=[
            pl.BlockSpec((BLK, H), lambda i: (i, 0)),
            pl.BlockSpec((8, H), lambda i: (0, 0)),
        ],
        out_shape=[
            jax.ShapeDtypeStruct((N, H), jnp.float32),
            jax.ShapeDtypeStruct((8, H), jnp.float32),
        ],
    )(h, a0, a1, W1, b1.reshape(1, H), W2, b2.reshape(1, H))


def _tc_bn_pool(m, stats, gamma, beta, batch3):
    def body(m_ref, s_ref, g_ref, b_ref, bt_ref, h_ref, p_ref):
        i = pl.program_id(0)
        mu = s_ref[0, :] * (1.0 / N)
        var = s_ref[1, :] * (1.0 / N) - mu * mu
        inv = lax.rsqrt(var + 1e-5)
        scale = inv * g_ref[0, :]
        shift = b_ref[0, :] - mu * scale
        hh = m_ref[...] * scale[None, :] + shift[None, :]
        h_ref[...] = hh
        bt = bt_ref[0, 0, :]
        onehot = (bt[:, None] == lax.broadcasted_iota(jnp.int32, (BLK, G), 1)
                  ).astype(jnp.float32)
        pool = lax.dot_general(onehot, hh, (((0,), (0,)), ((), ())),
                               preferred_element_type=jnp.float32)

        @pl.when(i == 0)
        def _():
            p_ref[...] = pool

        @pl.when(i != 0)
        def _():
            p_ref[...] += pool

    return pl.pallas_call(
        body,
        grid=(NBLK,),
        in_specs=[
            pl.BlockSpec((BLK, H), lambda i: (i, 0)),
            pl.BlockSpec((8, H), lambda i: (0, 0)),
            pl.BlockSpec((1, H), lambda i: (0, 0)),
            pl.BlockSpec((1, H), lambda i: (0, 0)),
            pl.BlockSpec((1, 1, BLK), lambda i: (i, 0, 0)),
        ],
        out_specs=[
            pl.BlockSpec((BLK, H), lambda i: (i, 0)),
            pl.BlockSpec((G, H), lambda i: (0, 0)),
        ],
        out_shape=[
            jax.ShapeDtypeStruct((N, H), jnp.float32),
            jax.ShapeDtypeStruct((G, H), jnp.float32),
        ],
    )(m, stats, gamma.reshape(1, H), beta.reshape(1, H), batch3)


def kernel(x, edge_index, batch,
           W1_0, b1_0, W2_0, b2_0, gamma_0, beta_0,
           W1_1, b1_1, W2_1, b2_1, gamma_1, beta_1,
           W1_2, b1_2, W2_2, b2_2, gamma_2, beta_2):
    src = edge_index[0].reshape(NW, EDGES_PER_TILE)
    dst = edge_index[1].reshape(NW, NCHUNKS, CHUNK)
    zeros = jnp.zeros((N_PAD, H), jnp.float32)
    batch3 = batch.reshape(NBLK, 1, BLK)
    layers = [(W1_0, b1_0, W2_0, b2_0, gamma_0, beta_0),
              (W1_1, b1_1, W2_1, b2_1, gamma_1, beta_1),
              (W1_2, b1_2, W2_2, b2_2, gamma_2, beta_2)]
    h = x
    xs = []
    pools = []
    for (W1, b1, W2, b2, g, b) in layers:
        agg = _sc_agg(h, src, dst, zeros)
        m, stats = _tc_mlp(h, agg[0], agg[1], W1, b1, W2, b2)
        h, pool = _tc_bn_pool(m, stats, g, b, batch3)
        xs.append(h)
        pools.append(pool)
    return (jnp.concatenate(pools, axis=1), jnp.concatenate(xs, axis=1))


# speedup vs baseline: 10.3872x; 1.0708x over previous
import functools

import jax
import jax.numpy as jnp
from jax import lax
from jax.experimental import pallas as pl
from jax.experimental.pallas import tpu as pltpu
from jax.experimental.pallas import tpu_sc as plsc

N = 10000
E = 320000
H = 128
G = 64

NC = 2
NS = 16
NW = NC * NS
EDGES_PER_TILE = E // NW
CHUNK = 80
NCHUNKS = EDGES_PER_TILE // CHUNK
ROWS_PER_SUB = 632
N_PAD = NS * ROWS_PER_SUB

BLK = 1000
NBLK = N // BLK


def _sc_agg(h, src, dst, zeros):
    mesh = plsc.VectorSubcoreMesh(core_axis_name="c", subcore_axis_name="s")

    NPAIRS = (NCHUNKS - 1) // 2

    @functools.partial(
        pl.kernel,
        mesh=mesh,
        out_type=jax.ShapeDtypeStruct((NC, N_PAD, H), jnp.float32),
        scratch_types=[
            pltpu.VMEM((EDGES_PER_TILE,), jnp.int32),
            pltpu.VMEM((NCHUNKS, CHUNK), jnp.int32),
            pltpu.VMEM((CHUNK, H), jnp.float32),
            pltpu.VMEM((CHUNK, H), jnp.float32),
            pltpu.VMEM_SHARED((N_PAD, H), jnp.float32),
            pltpu.SemaphoreType.DMA,
            pltpu.SemaphoreType.DMA,
            pltpu.SemaphoreType.DMA,
        ],
    )
    def k(h_hbm, src_hbm, dst_hbm, zeros_hbm, out_hbm, src_all, dst_all,
          rows0, rows1, acc, sem0, sem1, sem_idx):
        cid = lax.axis_index("c")
        sid = lax.axis_index("s")
        wid = sid * NC + cid
        row0 = sid * ROWS_PER_SUB
        pltpu.async_copy(src_hbm.at[wid], src_all, sem_idx)
        pltpu.async_copy(dst_hbm.at[wid], dst_all, sem_idx)
        pltpu.sync_copy(zeros_hbm.at[pl.ds(row0, ROWS_PER_SUB)],
                        acc.at[pl.ds(row0, ROWS_PER_SUB)])
        pltpu.make_async_copy(src_hbm.at[wid], src_all, sem_idx).wait()
        pltpu.make_async_copy(dst_hbm.at[wid], dst_all, sem_idx).wait()

        def gather_start(c, rows, sem):
            pltpu.async_copy(h_hbm.at[src_all.at[pl.ds(c * CHUNK, CHUNK)]],
                             rows, sem)

        def gather_wait(c, rows, sem):
            pltpu.make_async_copy(
                h_hbm.at[src_all.at[pl.ds(c * CHUNK, CHUNK)]], rows, sem
            ).wait()

        gather_start(0, rows0, sem0)
        plsc.subcore_barrier()

        @pl.loop(0, NPAIRS)
        def _(i):
            c = 2 * i
            gather_start(c + 1, rows1, sem1)
            gather_wait(c, rows0, sem0)
            pltpu.sync_copy(rows0, acc.at[dst_all.at[c]], add=True)
            gather_start(c + 2, rows0, sem0)
            gather_wait(c + 1, rows1, sem1)
            pltpu.sync_copy(rows1, acc.at[dst_all.at[c + 1]], add=True)

        gather_wait(NCHUNKS - 1, rows0, sem0)
        pltpu.sync_copy(rows0, acc.at[dst_all.at[NCHUNKS - 1]], add=True)

        plsc.subcore_barrier()
        pltpu.sync_copy(acc.at[pl.ds(row0, ROWS_PER_SUB)],
                        out_hbm.at[cid, pl.ds(row0, ROWS_PER_SUB)])

    return k(h, src, dst, zeros)


def _tc_layer(h, agg, W1, b1, W2, b2, gamma, beta, batch3):
    def body(h_ref, agg_ref, w1_ref, b1_ref, w2_ref, b2_ref, g_ref, bb_ref,
             batch_ref, h_out_ref, p_ref, m_buf, s_buf):
        ph = pl.program_id(0)
        i = pl.program_id(1)

        @pl.when(ph == 0)
        def _():
            x = h_ref[...] + agg_ref[0] + agg_ref[1]
            t = jnp.dot(x, w1_ref[...], preferred_element_type=jnp.float32)
            t = jnp.maximum(t + b1_ref[...], 0.0)
            t = jnp.dot(t, w2_ref[...], preferred_element_type=jnp.float32)
            t = t + b2_ref[...]
            m = jnp.where(t > 0.0, t, jnp.exp(jnp.minimum(t, 0.0)) - 1.0)
            m_buf[pl.ds(i * BLK, BLK), :] = m
            srow = jnp.sum(m, axis=0, keepdims=True)
            sqrow = jnp.sum(m * m, axis=0, keepdims=True)
            stats = jnp.concatenate(
                [srow, sqrow, jnp.zeros((6, H), jnp.float32)], axis=0)

            @pl.when(i == 0)
            def _():
                s_buf[...] = stats

            @pl.when(i != 0)
            def _():
                s_buf[...] += stats

        @pl.when(ph == 1)
        def _():
            mu = s_buf[0, :] * (1.0 / N)
            var = s_buf[1, :] * (1.0 / N) - mu * mu
            inv = lax.rsqrt(var + 1e-5)
            scale = inv * g_ref[0, :]
            shift = bb_ref[0, :] - mu * scale
            m = m_buf[pl.ds(i * BLK, BLK), :]
            hh = m * scale[None, :] + shift[None, :]
            h_out_ref[...] = hh
            bt = batch_ref[0, 0, :]
            onehot = (bt[:, None] == lax.broadcasted_iota(jnp.int32, (BLK, G), 1)
                      ).astype(jnp.float32)
            pool = lax.dot_general(onehot, hh, (((0,), (0,)), ((), ())),
                                   preferred_element_type=jnp.float32)

            @pl.when(i == 0)
            def _():
                p_ref[...] = pool

            @pl.when(i != 0)
            def _():
                p_ref[...] += pool

    return pl.pallas_call(
        body,
        grid=(2, NBLK),
        in_specs=[
            pl.BlockSpec((BLK, H), lambda p, i: (jnp.where(p == 0, i, 0), 0)),
            pl.BlockSpec((NC, BLK, H),
                         lambda p, i: (0, jnp.where(p == 0, i, 0), 0)),
            pl.BlockSpec((H, H), lambda p, i: (0, 0)),
            pl.BlockSpec((1, H), lambda p, i: (0, 0)),
            pl.BlockSpec((H, H), lambda p, i: (0, 0)),
            pl.BlockSpec((1, H), lambda p, i: (0, 0)),
            pl.BlockSpec((1, H), lambda p, i: (0, 0)),
            pl.BlockSpec((1, H), lambda p, i: (0, 0)),
            pl.BlockSpec((1, 1, BLK), lambda p, i: (jnp.where(p == 1, i, 0), 0, 0)),
        ],
        out_specs=[
            pl.BlockSpec((BLK, H), lambda p, i: (jnp.where(p == 1, i, 0), 0)),
            pl.BlockSpec((G, H), lambda p, i: (0, 0)),
        ],
        out_shape=[
            jax.ShapeDtypeStruct((N, H), jnp.float32),
            jax.ShapeDtypeStruct((G, H), jnp.float32),
        ],
        scratch_shapes=[
            pltpu.VMEM((N, H), jnp.float32),
            pltpu.VMEM((8, H), jnp.float32),
        ],
    )(h, agg, W1, b1.reshape(1, H), W2, b2.reshape(1, H),
      gamma.reshape(1, H), beta.reshape(1, H), batch3)


def kernel(x, edge_index, batch,
           W1_0, b1_0, W2_0, b2_0, gamma_0, beta_0,
           W1_1, b1_1, W2_1, b2_1, gamma_1, beta_1,
           W1_2, b1_2, W2_2, b2_2, gamma_2, beta_2):
    src = edge_index[0].reshape(NW, EDGES_PER_TILE)
    dst = edge_index[1].reshape(NW, NCHUNKS, CHUNK)
    zeros = jnp.zeros((N_PAD, H), jnp.float32)
    batch3 = batch.reshape(NBLK, 1, BLK)
    layers = [(W1_0, b1_0, W2_0, b2_0, gamma_0, beta_0),
              (W1_1, b1_1, W2_1, b2_1, gamma_1, beta_1),
              (W1_2, b1_2, W2_2, b2_2, gamma_2, beta_2)]
    h = x
    xs = []
    pools = []
    for (W1, b1, W2, b2, g, b) in layers:
        agg = _sc_agg(h, src, dst, zeros)
        h, pool = _tc_layer(h, agg, W1, b1, W2, b2, g, b, batch3)
        xs.append(h)
        pools.append(pool)
    return (jnp.concatenate(pools, axis=1), jnp.concatenate(xs, axis=1))


# speedup vs baseline: 10.6184x; 1.0223x over previous
import functools

import jax
import jax.numpy as jnp
from jax import lax
from jax.experimental import pallas as pl
from jax.experimental.pallas import tpu as pltpu
from jax.experimental.pallas import tpu_sc as plsc

N = 10000
E = 320000
H = 128
G = 64
L = 3

NC = 2
NS = 16
NW = NC * NS
EDGES_PER_TILE = E // NW
CHUNK = 80
NCHUNKS = EDGES_PER_TILE // CHUNK
NBUF = 2
ROWS_PER_SUB = 632
N_PAD = NS * ROWS_PER_SUB
ZITER = ROWS_PER_SUB // CHUNK
ZREM = ROWS_PER_SUB - ZITER * CHUNK

BLK = 1000
NBLK = N // BLK


def _sc_agg(h, src, dst):
    mesh = plsc.VectorSubcoreMesh(core_axis_name="c", subcore_axis_name="s")

    @functools.partial(
        pl.kernel,
        mesh=mesh,
        out_type=jax.ShapeDtypeStruct((NC, N_PAD, H), jnp.float32),
        scratch_types=[
            pltpu.VMEM((EDGES_PER_TILE,), jnp.int32),
            pltpu.VMEM((NCHUNKS, CHUNK), jnp.int32),
            pltpu.VMEM((CHUNK, H), jnp.float32),
            pltpu.VMEM((CHUNK, H), jnp.float32),
            pltpu.VMEM_SHARED((N_PAD, H), jnp.float32),
            pltpu.SemaphoreType.DMA,
            pltpu.SemaphoreType.DMA,
            pltpu.SemaphoreType.DMA,
        ],
    )
    def k(h_hbm, src_hbm, dst_hbm, out_hbm, src_all, dst_all, r0, r1,
          acc, s0, s1, sem_idx):
        cid = lax.axis_index("c")
        sid = lax.axis_index("s")
        wid = sid * NC + cid
        row0 = sid * ROWS_PER_SUB
        pltpu.async_copy(src_hbm.at[wid], src_all, sem_idx)
        pltpu.async_copy(dst_hbm.at[wid], dst_all, sem_idx)

        @pl.loop(0, CHUNK)
        def _(r):
            @pl.loop(0, H // 16)
            def _(j):
                r0[r, pl.ds(j * 16, 16)] = jnp.zeros((16,), jnp.float32)

        @pl.loop(0, ZITER)
        def _(j):
            pltpu.sync_copy(r0, acc.at[pl.ds(row0 + j * CHUNK, CHUNK)])

        pltpu.sync_copy(r0.at[pl.ds(0, ZREM)],
                        acc.at[pl.ds(row0 + ZITER * CHUNK, ZREM)])

        pltpu.make_async_copy(src_hbm.at[wid], src_all, sem_idx).wait()
        pltpu.make_async_copy(dst_hbm.at[wid], dst_all, sem_idx).wait()

        bufs = ((r0, s0), (r1, s1))

        def gstart(c, b):
            pltpu.async_copy(h_hbm.at[src_all.at[pl.ds(c * CHUNK, CHUNK)]],
                             bufs[b][0], bufs[b][1])

        def gwait(c, b):
            pltpu.make_async_copy(
                h_hbm.at[src_all.at[pl.ds(c * CHUNK, CHUNK)]],
                bufs[b][0], bufs[b][1]).wait()

        def scat(c, b):
            pltpu.sync_copy(bufs[b][0], acc.at[dst_all.at[c]], add=True)

        for b in range(NBUF):
            gstart(b, b)
        plsc.subcore_barrier()

        NLOOP = NCHUNKS // NBUF

        @pl.loop(0, NLOOP)
        def _(i):
            c0 = i * NBUF
            for b in range(NBUF):
                cc = c0 + b
                gwait(cc, b)
                scat(cc, b)
                nxt = cc + NBUF

                @pl.when(nxt < NCHUNKS)
                def _(nxt=nxt, b=b):
                    gstart(nxt, b)

        for b in range(NCHUNKS - NLOOP * NBUF):
            cc = NLOOP * NBUF + b
            gwait(cc, b)
            scat(cc, b)

        plsc.subcore_barrier()
        pltpu.sync_copy(acc.at[pl.ds(row0, ROWS_PER_SUB)],
                        out_hbm.at[cid, pl.ds(row0, ROWS_PER_SUB)])

    return k(h, src, dst)


def _tc_layer(hsrc, agg, W1, b1, W2, b2, gamma, beta, batch3, li):
    hcol = 0 if li == 0 else li - 1

    def body(h_ref, agg_ref, w1_ref, b1_ref, w2_ref, b2_ref, g_ref, bb_ref,
             batch_ref, xs_ref, p_ref, m_buf, s_buf):
        ph = pl.program_id(0)
        i = pl.program_id(1)

        @pl.when(ph == 0)
        def _():
            x = h_ref[...] + agg_ref[0] + agg_ref[1]
            t = jnp.dot(x.astype(jnp.bfloat16),
                        w1_ref[...].astype(jnp.bfloat16),
                        preferred_element_type=jnp.float32)
            t = jnp.maximum(t + b1_ref[...], 0.0)
            t = jnp.dot(t.astype(jnp.bfloat16),
                        w2_ref[...].astype(jnp.bfloat16),
                        preferred_element_type=jnp.float32)
            t = t + b2_ref[...]
            m = jnp.where(t > 0.0, t, jnp.exp(jnp.minimum(t, 0.0)) - 1.0)
            m_buf[pl.ds(i * BLK, BLK), :] = m
            srow = jnp.sum(m, axis=0, keepdims=True)
            sqrow = jnp.sum(m * m, axis=0, keepdims=True)
            stats = jnp.concatenate(
                [srow, sqrow, jnp.zeros((6, H), jnp.float32)], axis=0)

            @pl.when(i == 0)
            def _():
                s_buf[...] = stats

            @pl.when(i != 0)
            def _():
                s_buf[...] += stats

        @pl.when(ph == 1)
        def _():
            mu = s_buf[0, :] * (1.0 / N)
            var = s_buf[1, :] * (1.0 / N) - mu * mu
            inv = lax.rsqrt(var + 1e-5)
            scale = inv * g_ref[0, :]
            shift = bb_ref[0, :] - mu * scale
            m = m_buf[pl.ds(i * BLK, BLK), :]
            hh = m * scale[None, :] + shift[None, :]
            xs_ref[...] = hh
            bt = batch_ref[0, 0, :]
            onehot = (bt[:, None] == lax.broadcasted_iota(jnp.int32, (BLK, G), 1)
                      ).astype(jnp.float32)
            pool = lax.dot_general(onehot, hh, (((0,), (0,)), ((), ())),
                                   preferred_element_type=jnp.float32)

            @pl.when(i == 0)
            def _():
                p_ref[...] = pool

            @pl.when(i != 0)
            def _():
                p_ref[...] += pool

    return pl.pallas_call(
        body,
        grid=(2, NBLK),
        in_specs=[
            pl.BlockSpec((BLK, H),
                         lambda p, i: (jnp.where(p == 0, i, 0), hcol)),
            pl.BlockSpec((NC, BLK, H),
                         lambda p, i: (0, jnp.where(p == 0, i, 0), 0)),
            pl.BlockSpec((H, H), lambda p, i: (0, 0)),
            pl.BlockSpec((1, H), lambda p, i: (0, 0)),
            pl.BlockSpec((H, H), lambda p, i: (0, 0)),
            pl.BlockSpec((1, H), lambda p, i: (0, 0)),
            pl.BlockSpec((1, H), lambda p, i: (0, 0)),
            pl.BlockSpec((1, H), lambda p, i: (0, 0)),
            pl.BlockSpec((1, 1, BLK),
                         lambda p, i: (jnp.where(p == 1, i, 0), 0, 0)),
        ],
        out_specs=[
            pl.BlockSpec((BLK, H), lambda p, i: (jnp.where(p == 1, i, 0), li)),
            pl.BlockSpec((G, H), lambda p, i: (0, 0)),
        ],
        out_shape=[
            jax.ShapeDtypeStruct((N, L * H), jnp.float32),
            jax.ShapeDtypeStruct((G, H), jnp.float32),
        ],
        scratch_shapes=[
            pltpu.VMEM((N, H), jnp.float32),
            pltpu.VMEM((8, H), jnp.float32),
        ],
        input_output_aliases={} if li == 0 else {0: 0},
    )(hsrc, agg, W1, b1.reshape(1, H), W2, b2.reshape(1, H),
      gamma.reshape(1, H), beta.reshape(1, H), batch3)


def kernel(x, edge_index, batch,
           W1_0, b1_0, W2_0, b2_0, gamma_0, beta_0,
           W1_1, b1_1, W2_1, b2_1, gamma_1, beta_1,
           W1_2, b1_2, W2_2, b2_2, gamma_2, beta_2):
    src = edge_index[0].reshape(NW, EDGES_PER_TILE)
    dst = edge_index[1].reshape(NW, NCHUNKS, CHUNK)
    batch3 = batch.reshape(NBLK, 1, BLK)
    layers = [(W1_0, b1_0, W2_0, b2_0, gamma_0, beta_0),
              (W1_1, b1_1, W2_1, b2_1, gamma_1, beta_1),
              (W1_2, b1_2, W2_2, b2_2, gamma_2, beta_2)]
    pools = []
    hsrc = x
    xs = None
    for li, (W1, b1, W2, b2, g, b) in enumerate(layers):
        h = hsrc if li == 0 else lax.slice_in_dim(xs, (li - 1) * H, li * H, axis=1)
        agg = _sc_agg(h, src, dst)
        xs, pool = _tc_layer(hsrc if li == 0 else xs, agg, W1, b1, W2, b2,
                             g, b, batch3, li)
        hsrc = xs
        pools.append(pool)
    return (jnp.concatenate(pools, axis=1), xs)
